# baseline (speedup 1.0000x reference)
import functools

import jax
import jax.numpy as jnp
from jax import lax
from jax.experimental import pallas as pl
from jax.experimental.pallas import tpu as pltpu
from jax.experimental.pallas import tpu_sc as plsc

_N_RULES = 8
_EPS_C = 0.0001

_NC = 2
_NS = 16
_NW = _NC * _NS


@functools.lru_cache(maxsize=2)
def _eps_eager(batch: int, k: int):
    with jax.ensure_compile_time_eval():
        eps = jax.random.normal(
            jax.random.key(42), (batch, _N_RULES, k), jnp.float32)
        return jnp.transpose(eps, (1, 2, 0))


def _eps_const(batch: int, k: int):
    try:
        return _eps_eager(batch, k)
    except Exception:
        eps = jax.random.normal(
            jax.random.key(42), (batch, _N_RULES, k), jnp.float32)
        return jnp.transpose(eps, (1, 2, 0))


@functools.lru_cache(maxsize=2)
def _gather_fn(B: int, K: int, N: int):
    bpw = B // _NW
    mesh = plsc.VectorSubcoreMesh(
        core_axis_name="c", subcore_axis_name="s",
        num_cores=_NC, num_subcores=_NS)

    @functools.partial(
        pl.kernel, mesh=mesh,
        out_type=[jax.ShapeDtypeStruct((K, B), jnp.float32),
                  jax.ShapeDtypeStruct((K, B), jnp.float32)],
        scratch_types=[pltpu.VMEM((bpw,), jnp.int32),
                       pltpu.VMEM((bpw,), jnp.int32),
                       pltpu.VMEM((K, bpw), jnp.float32),
                       pltpu.VMEM((K, bpw), jnp.float32),
                       pltpu.SemaphoreType.DMA,
                       pltpu.SemaphoreType.DMA],
        compiler_params=pltpu.CompilerParams(use_tc_tiling_on_sc=False),
    )
    def gather(users, items, gu_t, gi_t, gu_o, gi_o,
               uidx, iidx, gu_v, gi_v, su, si):
        wid = lax.axis_index("s") * _NC + lax.axis_index("c")
        base = wid * bpw
        pltpu.sync_copy(users.at[pl.ds(base, bpw)], uidx)
        pltpu.sync_copy(items.at[pl.ds(base, bpw)], iidx)

        def issue(k, carry):
            pltpu.async_copy(gu_t.at[k].at[uidx], gu_v.at[k], su)
            pltpu.async_copy(gi_t.at[k].at[iidx], gi_v.at[k], si)
            return carry

        lax.fori_loop(0, K, issue, 0)
        pltpu.make_async_copy(gu_t.at[pl.ds(0, K), pl.ds(0, bpw)], gu_v, su).wait()
        pltpu.make_async_copy(gi_t.at[pl.ds(0, K), pl.ds(0, bpw)], gi_v, si).wait()
        pltpu.sync_copy(gu_v, gu_o.at[pl.ds(0, K), pl.ds(base, bpw)])
        pltpu.sync_copy(gi_v, gi_o.at[pl.ds(0, K), pl.ds(base, bpw)])

    return gather


def _dense_body(gut_ref, git_ref, epst_ref, gr_ref, gr3_ref, lvt_ref, xui_ref):
    gut = gut_ref[...]
    git = git_ref[...]
    gr = gr_ref[...]
    w = jnp.exp(0.5 * gr)
    scores = lax.dot_general(w, gut, (((1,), (0,)), ((), ())),
                             preferred_element_type=jnp.float32)
    m = jnp.max(scores, axis=0, keepdims=True)
    ex = jnp.exp(scores - m)
    s = ex / jnp.sum(ex, axis=0, keepdims=True)
    lvt = s[:, None, :] * gr3_ref[...]
    lvt_ref[...] = lvt
    noise = epst_ref[...] * jnp.exp(0.5 * lvt)
    gudot = jnp.sum(gut * git, axis=0)
    nd = jnp.sum(noise * git[None, :, :], axis=1)
    ands = nd + gudot[None, :]
    p = 1.0 - jax.nn.sigmoid(ands) + _EPS_C
    xui_ref[...] = 1.0 - jnp.exp(jnp.sum(jnp.log(p), axis=0))


@functools.lru_cache(maxsize=2)
def _dense_fn(B: int, K: int, blk: int, interpret: bool = False):
    grid = (B // blk,)
    return pl.pallas_call(
        _dense_body,
        grid=grid,
        in_specs=[
            pl.BlockSpec((K, blk), lambda i: (0, i)),
            pl.BlockSpec((K, blk), lambda i: (0, i)),
            pl.BlockSpec((_N_RULES, K, blk), lambda i: (0, 0, i)),
            pl.BlockSpec((_N_RULES, K), lambda i: (0, 0)),
            pl.BlockSpec((_N_RULES, K, 1), lambda i: (0, 0, 0)),
        ],
        out_specs=[
            pl.BlockSpec((_N_RULES, K, blk), lambda i: (0, 0, i)),
            pl.BlockSpec((blk,), lambda i: (i,)),
        ],
        out_shape=[
            jax.ShapeDtypeStruct((_N_RULES, K, B), jnp.float32),
            jax.ShapeDtypeStruct((B,), jnp.float32),
        ],
        interpret=interpret,
    )


def kernel(users, items, Gu_mean, Gr, Gi):
    B = users.shape[0]
    N, K = Gu_mean.shape
    users = users.astype(jnp.int32)
    items = items.astype(jnp.int32)
    Gu_t = Gu_mean.T
    Gi_t = Gi.T
    gu_t, gi_t = _gather_fn(B, K, N)(users, items, Gu_t, Gi_t)
    eps_t = _eps_const(B, K)
    lvt, xui = _dense_fn(B, K, 512)(gu_t, gi_t, eps_t, Gr, Gr[:, :, None])
    gu = gu_t.T
    lv = jnp.transpose(lvt, (2, 0, 1))
    return xui, gu, lv


# speedup vs baseline: 12.6457x; 12.6457x over previous
import functools

import jax
import jax.numpy as jnp
from jax import lax
from jax.experimental import pallas as pl
from jax.experimental.pallas import tpu as pltpu
from jax.experimental.pallas import tpu_sc as plsc

_N_RULES = 8
_EPS_C = 0.0001

_NC = 2
_NS = 16
_NW = _NC * _NS


@functools.lru_cache(maxsize=2)
def _eps_eager(batch: int, k: int):
    with jax.ensure_compile_time_eval():
        eps = jax.random.normal(
            jax.random.key(42), (batch, _N_RULES, k), jnp.float32)
        return jnp.transpose(eps, (1, 2, 0))


def _eps_const(batch: int, k: int):
    try:
        return _eps_eager(batch, k)
    except Exception:
        eps = jax.random.normal(
            jax.random.key(42), (batch, _N_RULES, k), jnp.float32)
        return jnp.transpose(eps, (1, 2, 0))


_ROUND = 32


@functools.lru_cache(maxsize=2)
def _gather_fn(B: int, K: int, N: int):
    bpw = B // _NW
    nro = bpw // _ROUND
    mesh = plsc.VectorSubcoreMesh(
        core_axis_name="c", subcore_axis_name="s",
        num_cores=_NC, num_subcores=_NS)

    @functools.partial(
        pl.kernel, mesh=mesh,
        out_type=jax.ShapeDtypeStruct((K * B,), jnp.float32),
        scratch_types=[pltpu.VMEM((bpw,), jnp.int32),
                       pltpu.VMEM((_ROUND, 8, K), jnp.float32),
                       pltpu.VMEM((K, bpw), jnp.float32),
                       pltpu.SemaphoreType.DMA],
        compiler_params=pltpu.CompilerParams(needs_layout_passes=False),
    )
    def gather(rows, tab, out, idxv, stage, outv, sem):
        wid = lax.axis_index("s") * _NC + lax.axis_index("c")
        base = wid * bpw
        lane = jnp.arange(16, dtype=jnp.int32)
        pltpu.sync_copy(rows.at[pl.ds(base, bpw)], idxv)

        def round_body(ro, carry):
            j0 = ro * _ROUND

            def issue(h, c):
                v16 = idxv[pl.ds(j0 + h * 16, 16)]
                for l in range(16):
                    r = lax.reduce_max(jnp.where(lane == l, v16, 0), axes=(0,))
                    r8 = pl.multiple_of((r >> 3) << 3, 8)
                    pltpu.async_copy(tab.at[pl.ds(r8, 8)],
                                     stage.at[h * 16 + l], sem)
                return c

            lax.fori_loop(0, _ROUND // 16, issue, 0)

            def drain(j, c):
                pltpu.make_async_copy(
                    tab.at[pl.ds(0, 8)], stage.at[0], sem).wait()
                return c

            lax.fori_loop(0, _ROUND, drain, 0)

            def extract(h, c):
                v16 = idxv[pl.ds(j0 + h * 16, 16)]
                for l in range(16):
                    r = lax.reduce_max(jnp.where(lane == l, v16, 0), axes=(0,))
                    rm = jnp.full((16,), r & 7, jnp.int32)
                    jj = jnp.full((16,), h * 16 + l, jnp.int32)
                    col = jnp.full((16,), j0 + h * 16 + l, jnp.int32)
                    for cc in range(K // 16):
                        va = plsc.load_gather(stage, [jj, rm, lane + 16 * cc])
                        plsc.store_scatter(outv, [lane + 16 * cc, col], va)
                return c

            lax.fori_loop(0, _ROUND // 16, extract, 0)
            return carry

        lax.fori_loop(0, nro, round_body, 0)

        def writeback(k, c):
            pltpu.sync_copy(outv.at[k], out.at[pl.ds(k * B + base, bpw)])
            return c

        lax.fori_loop(0, K, writeback, 0)

    return gather


def _dense_body(gut_ref, git_ref, epst_ref, gr_ref, gr3_ref, lvt_ref, xui_ref):
    gut = gut_ref[...]
    git = git_ref[...]
    gr = gr_ref[...]
    w = jnp.exp(0.5 * gr)
    scores = lax.dot_general(w, gut, (((1,), (0,)), ((), ())),
                             preferred_element_type=jnp.float32)
    m = jnp.max(scores, axis=0, keepdims=True)
    ex = jnp.exp(scores - m)
    s = ex / jnp.sum(ex, axis=0, keepdims=True)
    lvt = s[:, None, :] * gr3_ref[...]
    lvt_ref[...] = lvt
    noise = epst_ref[...] * jnp.exp(0.5 * lvt)
    gudot = jnp.sum(gut * git, axis=0)
    nd = jnp.sum(noise * git[None, :, :], axis=1)
    ands = nd + gudot[None, :]
    p = 1.0 - jax.nn.sigmoid(ands) + _EPS_C
    xui_ref[...] = 1.0 - jnp.exp(jnp.sum(jnp.log(p), axis=0))


@functools.lru_cache(maxsize=2)
def _dense_fn(B: int, K: int, blk: int, interpret: bool = False):
    grid = (B // blk,)
    return pl.pallas_call(
        _dense_body,
        grid=grid,
        in_specs=[
            pl.BlockSpec((K, blk), lambda i: (0, i)),
            pl.BlockSpec((K, blk), lambda i: (0, i)),
            pl.BlockSpec((_N_RULES, K, blk), lambda i: (0, 0, i)),
            pl.BlockSpec((_N_RULES, K), lambda i: (0, 0)),
            pl.BlockSpec((_N_RULES, K, 1), lambda i: (0, 0, 0)),
        ],
        out_specs=[
            pl.BlockSpec((_N_RULES, K, blk), lambda i: (0, 0, i)),
            pl.BlockSpec((blk,), lambda i: (i,)),
        ],
        out_shape=[
            jax.ShapeDtypeStruct((_N_RULES, K, B), jnp.float32),
            jax.ShapeDtypeStruct((B,), jnp.float32),
        ],
        interpret=interpret,
    )


def kernel(users, items, Gu_mean, Gr, Gi):
    B = users.shape[0]
    N, K = Gu_mean.shape
    users = users.astype(jnp.int32)
    items = items.astype(jnp.int32)
    g = _gather_fn(B, K, N)
    gu_t = g(users, Gu_mean).reshape(K, B)
    gi_t = g(items, Gi).reshape(K, B)
    eps_t = _eps_const(B, K)
    lvt, xui = _dense_fn(B, K, 512)(gu_t, gi_t, eps_t, Gr, Gr[:, :, None])
    gu = gu_t.T
    lv = jnp.transpose(lvt, (2, 0, 1))
    return xui, gu, lv


# speedup vs baseline: 19.4142x; 1.5352x over previous
import functools

import jax
import jax.numpy as jnp
from jax import lax
from jax.experimental import pallas as pl
from jax.experimental.pallas import tpu as pltpu
from jax.experimental.pallas import tpu_sc as plsc

_N_RULES = 8
_EPS_C = 0.0001

_NC = 2
_NS = 16
_NW = _NC * _NS


@functools.lru_cache(maxsize=2)
def _eps_eager(batch: int, k: int):
    with jax.ensure_compile_time_eval():
        eps = jax.random.normal(
            jax.random.key(42), (batch, _N_RULES, k), jnp.float32)
        return jnp.transpose(eps, (1, 2, 0))


def _eps_const(batch: int, k: int):
    try:
        return _eps_eager(batch, k)
    except Exception:
        eps = jax.random.normal(
            jax.random.key(42), (batch, _N_RULES, k), jnp.float32)
        return jnp.transpose(eps, (1, 2, 0))


_SLOTS = 3


@functools.lru_cache(maxsize=2)
def _gather_fn(B: int, K: int, N: int):
    bpw = B // _NW
    nt = K // 8
    mesh = plsc.VectorSubcoreMesh(
        core_axis_name="c", subcore_axis_name="s",
        num_cores=_NC, num_subcores=_NS)

    @functools.partial(
        pl.kernel, mesh=mesh,
        out_type=jax.ShapeDtypeStruct((K * B,), jnp.float32),
        scratch_types=[pltpu.VMEM((bpw,), jnp.int32),
                       pltpu.VMEM((_SLOTS, nt, 8, 128), jnp.float32),
                       pltpu.VMEM((K, bpw), jnp.float32),
                       pltpu.SemaphoreType.DMA],
        compiler_params=pltpu.CompilerParams(needs_layout_passes=False),
    )
    def gather(rows, tab, out, idxv, stage, outv, sem):
        wid = lax.axis_index("s") * _NC + lax.axis_index("c")
        base = wid * bpw
        lane = jnp.arange(16, dtype=jnp.int32)
        pltpu.sync_copy(rows.at[pl.ds(base, bpw)], idxv)

        def scalar_idx(j):
            v16 = idxv[pl.ds((j >> 4) << 4, 16)]
            return lax.reduce_max(jnp.where(lane == (j & 15), v16, 0),
                                  axes=(0,))

        def issue(j):
            r = scalar_idx(j)
            c0 = pl.multiple_of((r >> 7) << 7, 128)
            s = j - (j // _SLOTS) * _SLOTS
            for t in range(nt):
                pltpu.async_copy(tab.at[pl.ds(t * 8, 8), pl.ds(c0, 128)],
                                 stage.at[s, t], sem)

        def drain(j):
            for t in range(nt):
                pltpu.make_async_copy(
                    tab.at[pl.ds(0, 8), pl.ds(0, 128)],
                    stage.at[0, 0], sem).wait()

        def extract(j):
            r = scalar_idx(j)
            rcol = jnp.full((16,), r & 127, jnp.int32)
            col = jnp.full((16,), j, jnp.int32)
            s = j - (j // _SLOTS) * _SLOTS
            sv = jnp.full((16,), s, jnp.int32)
            for cc in range(K // 16):
                kv = lane + 16 * cc
                va = plsc.load_gather(stage, [sv, kv >> 3, kv & 7, rcol])
                plsc.store_scatter(outv, [kv, col], va)

        issue(0)
        issue(1)

        def body(j, carry):
            issue(j + 2)
            drain(j)
            extract(j)
            return carry

        lax.fori_loop(0, bpw - 2, body, 0)
        for j in (bpw - 2, bpw - 1):
            drain(j)
            extract(j)

        def writeback(k, c):
            pltpu.sync_copy(outv.at[k], out.at[pl.ds(k * B + base, bpw)])
            return c

        lax.fori_loop(0, K, writeback, 0)

    return gather


def _dense_body(gut_ref, git_ref, epst_ref, gr_ref, gr3_ref, lvt_ref, xui_ref):
    gut = gut_ref[...]
    git = git_ref[...]
    gr = gr_ref[...]
    w = jnp.exp(0.5 * gr)
    scores = lax.dot_general(w, gut, (((1,), (0,)), ((), ())),
                             preferred_element_type=jnp.float32)
    m = jnp.max(scores, axis=0, keepdims=True)
    ex = jnp.exp(scores - m)
    s = ex / jnp.sum(ex, axis=0, keepdims=True)
    lvt = s[:, None, :] * gr3_ref[...]
    lvt_ref[...] = lvt
    noise = epst_ref[...] * jnp.exp(0.5 * lvt)
    gudot = jnp.sum(gut * git, axis=0)
    nd = jnp.sum(noise * git[None, :, :], axis=1)
    ands = nd + gudot[None, :]
    p = 1.0 - jax.nn.sigmoid(ands) + _EPS_C
    xui_ref[...] = 1.0 - jnp.exp(jnp.sum(jnp.log(p), axis=0))


@functools.lru_cache(maxsize=2)
def _dense_fn(B: int, K: int, blk: int, interpret: bool = False):
    grid = (B // blk,)
    return pl.pallas_call(
        _dense_body,
        grid=grid,
        in_specs=[
            pl.BlockSpec((K, blk), lambda i: (0, i)),
            pl.BlockSpec((K, blk), lambda i: (0, i)),
            pl.BlockSpec((_N_RULES, K, blk), lambda i: (0, 0, i)),
            pl.BlockSpec((_N_RULES, K), lambda i: (0, 0)),
            pl.BlockSpec((_N_RULES, K, 1), lambda i: (0, 0, 0)),
        ],
        out_specs=[
            pl.BlockSpec((_N_RULES, K, blk), lambda i: (0, 0, i)),
            pl.BlockSpec((blk,), lambda i: (i,)),
        ],
        out_shape=[
            jax.ShapeDtypeStruct((_N_RULES, K, B), jnp.float32),
            jax.ShapeDtypeStruct((B,), jnp.float32),
        ],
        interpret=interpret,
    )


def kernel(users, items, Gu_mean, Gr, Gi):
    B = users.shape[0]
    N, K = Gu_mean.shape
    users = users.astype(jnp.int32)
    items = items.astype(jnp.int32)
    g = _gather_fn(B, K, N)
    gu_t = g(users, Gu_mean.T).reshape(K, B)
    gi_t = g(items, Gi.T).reshape(K, B)
    eps_t = _eps_const(B, K)
    lvt, xui = _dense_fn(B, K, 512)(gu_t, gi_t, eps_t, Gr, Gr[:, :, None])
    gu = gu_t.T
    lv = jnp.transpose(lvt, (2, 0, 1))
    return xui, gu, lv


# speedup vs baseline: 21.9257x; 1.1294x over previous
import functools

import jax
import jax.numpy as jnp
from jax import lax
from jax.experimental import pallas as pl
from jax.experimental.pallas import tpu as pltpu
from jax.experimental.pallas import tpu_sc as plsc

_N_RULES = 8
_EPS_C = 0.0001

_NC = 2
_NS = 16
_NW = _NC * _NS


@functools.lru_cache(maxsize=2)
def _eps_eager(batch: int, k: int):
    with jax.ensure_compile_time_eval():
        eps = jax.random.normal(
            jax.random.key(42), (batch, _N_RULES, k), jnp.float32)
        return jnp.transpose(eps, (1, 2, 0))


def _eps_const(batch: int, k: int):
    try:
        return _eps_eager(batch, k)
    except Exception:
        eps = jax.random.normal(
            jax.random.key(42), (batch, _N_RULES, k), jnp.float32)
        return jnp.transpose(eps, (1, 2, 0))


_SLOTS = 3


@functools.lru_cache(maxsize=2)
def _gather_fn(B: int, K: int, N: int):
    bpw = B // _NW
    nt = K // 8
    mesh = plsc.VectorSubcoreMesh(
        core_axis_name="c", subcore_axis_name="s",
        num_cores=_NC, num_subcores=_NS)

    @functools.partial(
        pl.kernel, mesh=mesh,
        out_type=jax.ShapeDtypeStruct((K * B,), jnp.float32),
        scratch_types=[pltpu.VMEM((bpw,), jnp.int32),
                       pltpu.VMEM((_SLOTS, nt, 8, 128), jnp.float32),
                       pltpu.VMEM((K, bpw), jnp.float32),
                       pltpu.SemaphoreType.DMA],
        compiler_params=pltpu.CompilerParams(needs_layout_passes=False),
    )
    def gather(rows, tab, out, idxv, stage, outv, sem):
        wid = lax.axis_index("s") * _NC + lax.axis_index("c")
        base = wid * bpw
        lane = jnp.arange(16, dtype=jnp.int32)
        pltpu.sync_copy(rows.at[pl.ds(base, bpw)], idxv)

        def scalar_idx(j):
            v16 = idxv[pl.ds((j >> 4) << 4, 16)]
            return lax.reduce_max(jnp.where(lane == (j & 15), v16, 0),
                                  axes=(0,))

        def issue(s, tc):
            c0 = pl.multiple_of(tc << 7, 128)
            for t in range(nt):
                pltpu.async_copy(tab.at[pl.ds(t * 8, 8), pl.ds(c0, 128)],
                                 stage.at[s, t], sem)

        def drain():
            for t in range(nt):
                pltpu.make_async_copy(
                    tab.at[pl.ds(0, 8), pl.ds(0, 128)],
                    stage.at[0, 0], sem).wait()

        def extract(j, s):
            r = scalar_idx(j)
            rcol = jnp.full((16,), r & 127, jnp.int32)
            col = jnp.full((16,), j, jnp.int32)
            sv = jnp.full((16,), s, jnp.int32)
            for cc in range(K // 16):
                kv = lane + 16 * cc
                va = plsc.load_gather(stage, [sv, kv >> 3, kv & 7, rcol])
                plsc.store_scatter(outv, [kv, col], va)

        def mod_slots(s):
            return s - (s // _SLOTS) * _SLOTS

        tc0 = scalar_idx(0) >> 7
        issue(0, tc0)
        tc1 = scalar_idx(1) >> 7
        f1 = tc1 != tc0
        s1 = jnp.where(f1, 1, 0)
        lax.cond(f1, lambda: issue(1, tc1), lambda: None)

        def body(j, carry):
            tc1, s1, f1, tc0, s0, f0 = carry
            tc2 = scalar_idx(j + 2) >> 7
            f2 = tc2 != tc1
            s2 = jnp.where(f2, mod_slots(s1 + 1), s1)
            lax.cond(f2, lambda: issue(s2, tc2), lambda: None)
            lax.cond(f0, drain, lambda: None)
            extract(j, s0)
            return (tc2, s2, f2, tc1, s1, f1)

        carry = lax.fori_loop(
            0, bpw - 2, body,
            (tc1, s1, f1, tc0, jnp.int32(0), jnp.bool_(True)))
        tc1, s1, f1, tc0, s0, f0 = carry
        lax.cond(f0, drain, lambda: None)
        extract(bpw - 2, s0)
        lax.cond(f1, drain, lambda: None)
        extract(bpw - 1, s1)

        def writeback(k, c):
            pltpu.sync_copy(outv.at[k], out.at[pl.ds(k * B + base, bpw)])
            return c

        lax.fori_loop(0, K, writeback, 0)

    return gather


def _dense_body(gut_ref, git_ref, epst_ref, gr_ref, gr3_ref, lvt_ref, xui_ref):
    gut = gut_ref[...]
    git = git_ref[...]
    gr = gr_ref[...]
    w = jnp.exp(0.5 * gr)
    scores = lax.dot_general(w, gut, (((1,), (0,)), ((), ())),
                             preferred_element_type=jnp.float32)
    m = jnp.max(scores, axis=0, keepdims=True)
    ex = jnp.exp(scores - m)
    s = ex / jnp.sum(ex, axis=0, keepdims=True)
    lvt = s[:, None, :] * gr3_ref[...]
    lvt_ref[...] = lvt
    noise = epst_ref[...] * jnp.exp(0.5 * lvt)
    gudot = jnp.sum(gut * git, axis=0)
    nd = jnp.sum(noise * git[None, :, :], axis=1)
    ands = nd + gudot[None, :]
    p = 1.0 - jax.nn.sigmoid(ands) + _EPS_C
    xui_ref[...] = 1.0 - jnp.exp(jnp.sum(jnp.log(p), axis=0))


@functools.lru_cache(maxsize=2)
def _dense_fn(B: int, K: int, blk: int, interpret: bool = False):
    grid = (B // blk,)
    return pl.pallas_call(
        _dense_body,
        grid=grid,
        in_specs=[
            pl.BlockSpec((K, blk), lambda i: (0, i)),
            pl.BlockSpec((K, blk), lambda i: (0, i)),
            pl.BlockSpec((_N_RULES, K, blk), lambda i: (0, 0, i)),
            pl.BlockSpec((_N_RULES, K), lambda i: (0, 0)),
            pl.BlockSpec((_N_RULES, K, 1), lambda i: (0, 0, 0)),
        ],
        out_specs=[
            pl.BlockSpec((_N_RULES, K, blk), lambda i: (0, 0, i)),
            pl.BlockSpec((blk,), lambda i: (i,)),
        ],
        out_shape=[
            jax.ShapeDtypeStruct((_N_RULES, K, B), jnp.float32),
            jax.ShapeDtypeStruct((B,), jnp.float32),
        ],
        interpret=interpret,
    )


def kernel(users, items, Gu_mean, Gr, Gi):
    B = users.shape[0]
    N, K = Gu_mean.shape
    users = users.astype(jnp.int32)
    items = items.astype(jnp.int32)
    iota = jnp.arange(B, dtype=jnp.int32)
    g = _gather_fn(B, K, N)
    outs = []
    for idx, tab in ((users, Gu_mean), (items, Gi)):
        s_idx, order = lax.sort_key_val(idx, iota)
        got = g(s_idx, tab.T).reshape(K, B)
        inv = jnp.zeros((B,), jnp.int32).at[order].set(iota)
        outs.append(jnp.take(got, inv, axis=1))
    gu_t, gi_t = outs
    eps_t = _eps_const(B, K)
    lvt, xui = _dense_fn(B, K, 512)(gu_t, gi_t, eps_t, Gr, Gr[:, :, None])
    gu = gu_t.T
    lv = jnp.transpose(lvt, (2, 0, 1))
    return xui, gu, lv


# speedup vs baseline: 22.0011x; 1.0034x over previous
import functools

import jax
import jax.numpy as jnp
from jax import lax
from jax.experimental import pallas as pl
from jax.experimental.pallas import tpu as pltpu
from jax.experimental.pallas import tpu_sc as plsc

_N_RULES = 8
_EPS_C = 0.0001

_NC = 2
_NS = 16
_NW = _NC * _NS


@functools.lru_cache(maxsize=2)
def _eps_eager(batch: int, k: int):
    with jax.ensure_compile_time_eval():
        eps = jax.random.normal(
            jax.random.key(42), (batch, _N_RULES, k), jnp.float32)
        return jnp.transpose(eps, (1, 2, 0))


def _eps_const(batch: int, k: int):
    try:
        return _eps_eager(batch, k)
    except Exception:
        eps = jax.random.normal(
            jax.random.key(42), (batch, _N_RULES, k), jnp.float32)
        return jnp.transpose(eps, (1, 2, 0))


_SLOTS = 3


@functools.lru_cache(maxsize=2)
def _gather_fn(B: int, K: int, N: int):
    bpw = B // _NW
    nt = K // 8
    mesh = plsc.VectorSubcoreMesh(
        core_axis_name="c", subcore_axis_name="s",
        num_cores=_NC, num_subcores=_NS)

    @functools.partial(
        pl.kernel, mesh=mesh,
        out_type=jax.ShapeDtypeStruct((K * B,), jnp.float32),
        scratch_types=[pltpu.VMEM((bpw,), jnp.int32),
                       pltpu.VMEM((_SLOTS, nt, 8, 128), jnp.float32),
                       pltpu.VMEM((K, bpw), jnp.float32),
                       pltpu.SemaphoreType.DMA],
        compiler_params=pltpu.CompilerParams(needs_layout_passes=False),
    )
    def gather(rows, tab, out, idxv, stage, outv, sem):
        wid = lax.axis_index("s") * _NC + lax.axis_index("c")
        base = wid * bpw
        lane = jnp.arange(16, dtype=jnp.int32)
        pltpu.sync_copy(rows.at[pl.ds(base, bpw)], idxv)

        def scalar_idx(j):
            v16 = idxv[pl.ds((j >> 4) << 4, 16)]
            return lax.reduce_max(jnp.where(lane == (j & 15), v16, 0),
                                  axes=(0,))

        def issue(s, tc):
            c0 = pl.multiple_of(tc << 7, 128)
            for t in range(nt):
                pltpu.async_copy(tab.at[pl.ds(t * 8, 8), pl.ds(c0, 128)],
                                 stage.at[s, t], sem)

        def drain():
            for t in range(nt):
                pltpu.make_async_copy(
                    tab.at[pl.ds(0, 8), pl.ds(0, 128)],
                    stage.at[0, 0], sem).wait()

        def extract(j, s):
            r = scalar_idx(j)
            rcol = jnp.full((16,), r & 127, jnp.int32)
            col = jnp.full((16,), j, jnp.int32)
            sv = jnp.full((16,), s, jnp.int32)
            for cc in range(K // 16):
                kv = lane + 16 * cc
                va = plsc.load_gather(stage, [sv, kv >> 3, kv & 7, rcol])
                plsc.store_scatter(outv, [kv, col], va)

        def mod_slots(s):
            return s - (s // _SLOTS) * _SLOTS

        tc0 = scalar_idx(0) >> 7
        issue(0, tc0)
        tc1 = scalar_idx(1) >> 7
        f1 = tc1 != tc0
        s1 = jnp.where(f1, 1, 0)
        lax.cond(f1, lambda: issue(1, tc1), lambda: None)

        def body(j, carry):
            tc1, s1, f1, tc0, s0, f0 = carry
            tc2 = scalar_idx(j + 2) >> 7
            f2 = tc2 != tc1
            s2 = jnp.where(f2, mod_slots(s1 + 1), s1)
            lax.cond(f2, lambda: issue(s2, tc2), lambda: None)
            lax.cond(f0, drain, lambda: None)
            extract(j, s0)
            return (tc2, s2, f2, tc1, s1, f1)

        carry = lax.fori_loop(
            0, bpw - 2, body,
            (tc1, s1, f1, tc0, jnp.int32(0), jnp.bool_(True)))
        tc1, s1, f1, tc0, s0, f0 = carry
        lax.cond(f0, drain, lambda: None)
        extract(bpw - 2, s0)
        lax.cond(f1, drain, lambda: None)
        extract(bpw - 1, s1)

        def writeback(k, c):
            pltpu.sync_copy(outv.at[k], out.at[pl.ds(k * B + base, bpw)])
            return c

        lax.fori_loop(0, K, writeback, 0)

    return gather


def _dense_body(gut_ref, git_ref, epst_ref, gr_ref, gr3_ref, lvt_ref, xui_ref):
    gut = gut_ref[...]
    git = git_ref[...]
    gr = gr_ref[...]
    w = jnp.exp(0.5 * gr)
    scores = lax.dot_general(w, gut, (((1,), (0,)), ((), ())),
                             preferred_element_type=jnp.float32)
    m = jnp.max(scores, axis=0, keepdims=True)
    ex = jnp.exp(scores - m)
    s = ex / jnp.sum(ex, axis=0, keepdims=True)
    lvt = s[:, None, :] * gr3_ref[...]
    lvt_ref[...] = lvt
    noise = epst_ref[...] * jnp.exp(0.5 * lvt)
    gudot = jnp.sum(gut * git, axis=0)
    nd = jnp.sum(noise * git[None, :, :], axis=1)
    ands = nd + gudot[None, :]
    p = 1.0 - jax.nn.sigmoid(ands) + _EPS_C
    xui_ref[...] = 1.0 - jnp.exp(jnp.sum(jnp.log(p), axis=0))


@functools.lru_cache(maxsize=2)
def _dense_fn(B: int, K: int, blk: int, interpret: bool = False):
    grid = (B // blk,)
    return pl.pallas_call(
        _dense_body,
        grid=grid,
        in_specs=[
            pl.BlockSpec((K, blk), lambda i: (0, i)),
            pl.BlockSpec((K, blk), lambda i: (0, i)),
            pl.BlockSpec((_N_RULES, K, blk), lambda i: (0, 0, i)),
            pl.BlockSpec((_N_RULES, K), lambda i: (0, 0)),
            pl.BlockSpec((_N_RULES, K, 1), lambda i: (0, 0, 0)),
        ],
        out_specs=[
            pl.BlockSpec((_N_RULES, K, blk), lambda i: (0, 0, i)),
            pl.BlockSpec((blk,), lambda i: (i,)),
        ],
        out_shape=[
            jax.ShapeDtypeStruct((_N_RULES, K, B), jnp.float32),
            jax.ShapeDtypeStruct((B,), jnp.float32),
        ],
        interpret=interpret,
    )


def kernel(users, items, Gu_mean, Gr, Gi):
    B = users.shape[0]
    N, K = Gu_mean.shape
    users = users.astype(jnp.int32)
    items = items.astype(jnp.int32)
    iota = jnp.arange(B, dtype=jnp.int32)
    g = _gather_fn(B, K, N)
    outs = []
    for idx, tab in ((users, Gu_mean), (items, Gi)):
        s_idx, order = lax.sort_key_val(idx, iota)
        got = g(s_idx, tab.T).reshape(K, B)
        _, inv = lax.sort_key_val(order, iota)
        outs.append(jnp.take(got, inv, axis=1))
    gu_t, gi_t = outs
    eps_t = _eps_const(B, K)
    lvt, xui = _dense_fn(B, K, 512)(gu_t, gi_t, eps_t, Gr, Gr[:, :, None])
    gu = gu_t.T
    lv = jnp.transpose(lvt, (2, 0, 1))
    return xui, gu, lv


# speedup vs baseline: 22.4346x; 1.0197x over previous
import functools

import jax
import jax.numpy as jnp
from jax import lax
from jax.experimental import pallas as pl
from jax.experimental.pallas import tpu as pltpu
from jax.experimental.pallas import tpu_sc as plsc

_N_RULES = 8
_EPS_C = 0.0001

_NC = 2
_NS = 16
_NW = _NC * _NS


@functools.lru_cache(maxsize=2)
def _eps_eager(batch: int, k: int):
    with jax.ensure_compile_time_eval():
        eps = jax.random.normal(
            jax.random.key(42), (batch, _N_RULES, k), jnp.float32)
        return jnp.transpose(eps, (1, 2, 0))


def _eps_const(batch: int, k: int):
    try:
        return _eps_eager(batch, k)
    except Exception:
        eps = jax.random.normal(
            jax.random.key(42), (batch, _N_RULES, k), jnp.float32)
        return jnp.transpose(eps, (1, 2, 0))


_SLOTS = 3


@functools.lru_cache(maxsize=2)
def _gather_fn(B: int, K: int, N: int):
    bpw = B // _NW
    nt = K // 8
    mesh = plsc.VectorSubcoreMesh(
        core_axis_name="c", subcore_axis_name="s",
        num_cores=_NC, num_subcores=_NS)

    @functools.partial(
        pl.kernel, mesh=mesh,
        out_type=jax.ShapeDtypeStruct((K * B,), jnp.float32),
        scratch_types=[pltpu.VMEM((bpw,), jnp.int32),
                       pltpu.VMEM((_SLOTS, nt, 8, 128), jnp.float32),
                       pltpu.VMEM((K, bpw), jnp.float32),
                       pltpu.SemaphoreType.DMA],
        compiler_params=pltpu.CompilerParams(needs_layout_passes=False),
    )
    def gather(rows, tab, out, idxv, stage, outv, sem):
        wid = lax.axis_index("s") * _NC + lax.axis_index("c")
        base = wid * bpw
        lane = jnp.arange(16, dtype=jnp.int32)
        pltpu.sync_copy(rows.at[pl.ds(base, bpw)], idxv)

        def scalar_idx(j):
            v16 = idxv[pl.ds((j >> 4) << 4, 16)]
            return lax.reduce_max(jnp.where(lane == (j & 15), v16, 0),
                                  axes=(0,))

        def issue(s, tc):
            c0 = pl.multiple_of(tc << 7, 128)
            for t in range(nt):
                pltpu.async_copy(tab.at[pl.ds(t * 8, 8), pl.ds(c0, 128)],
                                 stage.at[s, t], sem)

        def drain():
            for t in range(nt):
                pltpu.make_async_copy(
                    tab.at[pl.ds(0, 8), pl.ds(0, 128)],
                    stage.at[0, 0], sem).wait()

        def extract(j, s, r):
            rcol = jnp.full((16,), r & 127, jnp.int32)
            col = jnp.full((16,), j, jnp.int32)
            sv = jnp.full((16,), s, jnp.int32)
            for cc in range(K // 16):
                kv = lane + 16 * cc
                va = plsc.load_gather(stage, [sv, kv >> 3, kv & 7, rcol])
                plsc.store_scatter(outv, [kv, col], va)

        def mod_slots(s):
            return s - (s // _SLOTS) * _SLOTS

        r0 = scalar_idx(0)
        issue(0, r0 >> 7)
        r1 = scalar_idx(1)
        f1 = (r1 >> 7) != (r0 >> 7)
        s1 = jnp.where(f1, jnp.int32(1), jnp.int32(0))
        lax.cond(f1, lambda: issue(1, r1 >> 7), lambda: None)

        def body(j, carry):
            r1, s1, f1, r0, s0, f0 = carry
            r2 = scalar_idx(j + 2)
            f2 = (r2 >> 7) != (r1 >> 7)
            s2 = jnp.where(f2, mod_slots(s1 + 1), s1)
            lax.cond(f2, lambda: issue(s2, r2 >> 7), lambda: None)
            lax.cond(f0, drain, lambda: None)
            extract(j, s0, r0)
            return (r2, s2, f2, r1, s1, f1)

        carry = lax.fori_loop(
            0, bpw - 2, body,
            (r1, s1, f1, r0, jnp.int32(0), jnp.bool_(True)))
        r1, s1, f1, r0, s0, f0 = carry
        lax.cond(f0, drain, lambda: None)
        extract(bpw - 2, s0, r0)
        lax.cond(f1, drain, lambda: None)
        extract(bpw - 1, s1, r1)

        def writeback(k, c):
            pltpu.sync_copy(outv.at[k], out.at[pl.ds(k * B + base, bpw)])
            return c

        lax.fori_loop(0, K, writeback, 0)

    return gather


def _dense_body(gut_ref, git_ref, epst_ref, gr_ref, gr3_ref, lvt_ref, xui_ref):
    gut = gut_ref[...]
    git = git_ref[...]
    gr = gr_ref[...]
    w = jnp.exp(0.5 * gr)
    scores = lax.dot_general(w, gut, (((1,), (0,)), ((), ())),
                             preferred_element_type=jnp.float32)
    m = jnp.max(scores, axis=0, keepdims=True)
    ex = jnp.exp(scores - m)
    s = ex / jnp.sum(ex, axis=0, keepdims=True)
    lvt = s[:, None, :] * gr3_ref[...]
    lvt_ref[...] = lvt
    noise = epst_ref[...] * jnp.exp(0.5 * lvt)
    gudot = jnp.sum(gut * git, axis=0)
    nd = jnp.sum(noise * git[None, :, :], axis=1)
    ands = nd + gudot[None, :]
    p = 1.0 - jax.nn.sigmoid(ands) + _EPS_C
    xui_ref[...] = 1.0 - jnp.exp(jnp.sum(jnp.log(p), axis=0))


@functools.lru_cache(maxsize=2)
def _dense_fn(B: int, K: int, blk: int, interpret: bool = False):
    grid = (B // blk,)
    return pl.pallas_call(
        _dense_body,
        grid=grid,
        in_specs=[
            pl.BlockSpec((K, blk), lambda i: (0, i)),
            pl.BlockSpec((K, blk), lambda i: (0, i)),
            pl.BlockSpec((_N_RULES, K, blk), lambda i: (0, 0, i)),
            pl.BlockSpec((_N_RULES, K), lambda i: (0, 0)),
            pl.BlockSpec((_N_RULES, K, 1), lambda i: (0, 0, 0)),
        ],
        out_specs=[
            pl.BlockSpec((_N_RULES, K, blk), lambda i: (0, 0, i)),
            pl.BlockSpec((blk,), lambda i: (i,)),
        ],
        out_shape=[
            jax.ShapeDtypeStruct((_N_RULES, K, B), jnp.float32),
            jax.ShapeDtypeStruct((B,), jnp.float32),
        ],
        interpret=interpret,
    )


def kernel(users, items, Gu_mean, Gr, Gi):
    B = users.shape[0]
    N, K = Gu_mean.shape
    users = users.astype(jnp.int32)
    items = items.astype(jnp.int32)
    iota = jnp.arange(B, dtype=jnp.int32)
    g = _gather_fn(B, K, N)
    outs = []
    for idx, tab in ((users, Gu_mean), (items, Gi)):
        s_idx, order = lax.sort_key_val(idx, iota)
        got = g(s_idx, tab.T).reshape(K, B)
        _, inv = lax.sort_key_val(order, iota)
        outs.append(jnp.take(got, inv, axis=1))
    gu_t, gi_t = outs
    eps_t = _eps_const(B, K)
    lvt, xui = _dense_fn(B, K, 512)(gu_t, gi_t, eps_t, Gr, Gr[:, :, None])
    gu = gu_t.T
    lv = jnp.transpose(lvt, (2, 0, 1))
    return xui, gu, lv


# speedup vs baseline: 27.9503x; 1.2459x over previous
import functools

import jax
import jax.numpy as jnp
from jax import lax
from jax.experimental import pallas as pl
from jax.experimental.pallas import tpu as pltpu
from jax.experimental.pallas import tpu_sc as plsc

_N_RULES = 8
_EPS_C = 0.0001

_NC = 2
_NS = 16
_NW = _NC * _NS


@functools.lru_cache(maxsize=2)
def _eps_eager(batch: int, k: int):
    with jax.ensure_compile_time_eval():
        eps = jax.random.normal(
            jax.random.key(42), (batch, _N_RULES, k), jnp.float32)
        return jnp.transpose(eps, (1, 2, 0))


def _eps_const(batch: int, k: int):
    try:
        return _eps_eager(batch, k)
    except Exception:
        eps = jax.random.normal(
            jax.random.key(42), (batch, _N_RULES, k), jnp.float32)
        return jnp.transpose(eps, (1, 2, 0))


_SLOTS = 5
_DEPTH = 4


@functools.lru_cache(maxsize=2)
def _gather_fn(B: int, K: int, N: int):
    bpw = B // _NW
    nt = K // 8
    mesh = plsc.VectorSubcoreMesh(
        core_axis_name="c", subcore_axis_name="s",
        num_cores=_NC, num_subcores=_NS)

    @functools.partial(
        pl.kernel, mesh=mesh,
        out_type=jax.ShapeDtypeStruct((K * B,), jnp.float32),
        scratch_types=[pltpu.VMEM((bpw,), jnp.int32),
                       pltpu.VMEM((_SLOTS, nt, 8, 128), jnp.float32),
                       pltpu.VMEM((K, bpw // 2), jnp.float32),
                       pltpu.SemaphoreType.DMA],
        compiler_params=pltpu.CompilerParams(needs_layout_passes=False),
    )
    def gather(rows, tab, out, idxv, stage, outv, sem):
        wid = lax.axis_index("s") * _NC + lax.axis_index("c")
        base = wid * bpw
        lane = jnp.arange(16, dtype=jnp.int32)
        pltpu.sync_copy(rows.at[pl.ds(base, bpw)], idxv)

        def scalar_idx(j):
            v16 = idxv[pl.ds((j >> 4) << 4, 16)]
            return lax.reduce_max(jnp.where(lane == (j & 15), v16, 0),
                                  axes=(0,))

        def issue(s, tc):
            c0 = pl.multiple_of(tc << 7, 128)
            for t in range(nt):
                pltpu.async_copy(tab.at[pl.ds(t * 8, 8), pl.ds(c0, 128)],
                                 stage.at[s, t], sem)

        def drain():
            for t in range(nt):
                pltpu.make_async_copy(
                    tab.at[pl.ds(0, 8), pl.ds(0, 128)],
                    stage.at[0, 0], sem).wait()

        half = bpw // 2

        def extract(j, s, r):
            rcol = jnp.full((16,), r & 127, jnp.int32)
            col = jnp.full((16,), j & (half - 1), jnp.int32)
            sv = jnp.full((16,), s, jnp.int32)
            for cc in range(K // 16):
                kv = lane + 16 * cc
                va = plsc.load_gather(stage, [sv, kv >> 3, kv & 7, rcol])
                plsc.store_scatter(outv, [kv, col], va)

        def mod_slots(s):
            return s - (s // _SLOTS) * _SLOTS

        def writeback(phase):
            def wb(k, c):
                pltpu.sync_copy(outv.at[k],
                                out.at[pl.ds(k * B + base + phase * half, half)])
                return c
            lax.fori_loop(0, K, wb, 0)

        chain = []
        r_p, s_p, f_p = None, jnp.int32(0), None
        for j in range(_DEPTH):
            r_j = scalar_idx(j)
            if j == 0:
                f_j, s_j = jnp.bool_(True), jnp.int32(0)
            else:
                f_j = (r_j >> 7) != (r_p >> 7)
                s_j = jnp.where(f_j, mod_slots(s_p + 1), s_p)
            lax.cond(f_j, lambda r=r_j, s=s_j: issue(s, r >> 7), lambda: None)
            chain.insert(0, (r_j, s_j, f_j))
            r_p, s_p, f_p = r_j, s_j, f_j

        def body(j, carry):
            (r3, s3, f3), (r2, s2, f2), (r1, s1, f1), (r0, s0, f0) = (
                carry[0:3], carry[3:6], carry[6:9], carry[9:12])
            rn = scalar_idx(j + _DEPTH)
            fn = (rn >> 7) != (r3 >> 7)
            sn = jnp.where(fn, mod_slots(s3 + 1), s3)
            lax.cond(fn, lambda: issue(sn, rn >> 7), lambda: None)
            lax.cond(f0, drain, lambda: None)
            extract(j, s0, r0)
            lax.cond(j == half - 1, lambda: writeback(0), lambda: None)
            return (rn, sn, fn, r3, s3, f3, r2, s2, f2, r1, s1, f1)

        init = tuple(x for t in chain for x in t)
        carry = lax.fori_loop(0, bpw - _DEPTH, body, init)
        tail = [carry[9:12], carry[6:9], carry[3:6], carry[0:3]]
        for i, (r_j, s_j, f_j) in enumerate(tail):
            lax.cond(f_j, drain, lambda: None)
            extract(bpw - _DEPTH + i, s_j, r_j)
        writeback(1)

    return gather


def _dense_body(gut_ref, git_ref, epst_ref, gr_ref, gr3_ref, lvt_ref, xui_ref):
    gut = gut_ref[...]
    git = git_ref[...]
    gr = gr_ref[...]
    w = jnp.exp(0.5 * gr)
    scores = lax.dot_general(w, gut, (((1,), (0,)), ((), ())),
                             preferred_element_type=jnp.float32)
    m = jnp.max(scores, axis=0, keepdims=True)
    ex = jnp.exp(scores - m)
    s = ex / jnp.sum(ex, axis=0, keepdims=True)
    lvt = s[:, None, :] * gr3_ref[...]
    lvt_ref[...] = lvt
    noise = epst_ref[...] * jnp.exp(0.5 * lvt)
    gudot = jnp.sum(gut * git, axis=0)
    nd = jnp.sum(noise * git[None, :, :], axis=1)
    ands = nd + gudot[None, :]
    p = 1.0 - jax.nn.sigmoid(ands) + _EPS_C
    xui_ref[...] = 1.0 - jnp.exp(jnp.sum(jnp.log(p), axis=0))


@functools.lru_cache(maxsize=2)
def _dense_fn(B: int, K: int, blk: int, interpret: bool = False):
    grid = (B // blk,)
    return pl.pallas_call(
        _dense_body,
        grid=grid,
        in_specs=[
            pl.BlockSpec((K, blk), lambda i: (0, i)),
            pl.BlockSpec((K, blk), lambda i: (0, i)),
            pl.BlockSpec((_N_RULES, K, blk), lambda i: (0, 0, i)),
            pl.BlockSpec((_N_RULES, K), lambda i: (0, 0)),
            pl.BlockSpec((_N_RULES, K, 1), lambda i: (0, 0, 0)),
        ],
        out_specs=[
            pl.BlockSpec((_N_RULES, K, blk), lambda i: (0, 0, i)),
            pl.BlockSpec((blk,), lambda i: (i,)),
        ],
        out_shape=[
            jax.ShapeDtypeStruct((_N_RULES, K, B), jnp.float32),
            jax.ShapeDtypeStruct((B,), jnp.float32),
        ],
        interpret=interpret,
    )


def kernel(users, items, Gu_mean, Gr, Gi):
    B = users.shape[0]
    N, K = Gu_mean.shape
    users = users.astype(jnp.int32)
    items = items.astype(jnp.int32)
    iota = jnp.arange(B, dtype=jnp.int32)
    g = _gather_fn(B, K, N)
    outs = []
    for idx, tab in ((users, Gu_mean), (items, Gi)):
        s_idx, order = lax.sort_key_val(idx, iota)
        got = g(s_idx, tab.T).reshape(K, B)
        _, inv = lax.sort_key_val(order, iota)
        outs.append(jnp.take(got, inv, axis=1))
    gu_t, gi_t = outs
    eps_t = _eps_const(B, K)
    lvt, xui = _dense_fn(B, K, 512)(gu_t, gi_t, eps_t, Gr, Gr[:, :, None])
    gu = gu_t.T
    lv = jnp.transpose(lvt, (2, 0, 1))
    return xui, gu, lv


# speedup vs baseline: 28.8009x; 1.0304x over previous
import functools

import jax
import jax.numpy as jnp
from jax import lax
from jax.experimental import pallas as pl
from jax.experimental.pallas import tpu as pltpu
from jax.experimental.pallas import tpu_sc as plsc

_N_RULES = 8
_EPS_C = 0.0001

_NC = 2
_NS = 16
_NW = _NC * _NS


@functools.lru_cache(maxsize=2)
def _eps_eager(batch: int, k: int):
    with jax.ensure_compile_time_eval():
        eps = jax.random.normal(
            jax.random.key(42), (batch, _N_RULES, k), jnp.float32)
        return jnp.transpose(eps, (1, 2, 0))


def _eps_const(batch: int, k: int):
    try:
        return _eps_eager(batch, k)
    except Exception:
        eps = jax.random.normal(
            jax.random.key(42), (batch, _N_RULES, k), jnp.float32)
        return jnp.transpose(eps, (1, 2, 0))


_SLOTS = 6
_DEPTH = 5
_PHASES = 4


@functools.lru_cache(maxsize=2)
def _gather_fn(B: int, K: int, N: int):
    bpw = B // _NW
    nt = K // 8
    mesh = plsc.VectorSubcoreMesh(
        core_axis_name="c", subcore_axis_name="s",
        num_cores=_NC, num_subcores=_NS)

    @functools.partial(
        pl.kernel, mesh=mesh,
        out_type=jax.ShapeDtypeStruct((K * B,), jnp.float32),
        scratch_types=[pltpu.VMEM((bpw,), jnp.int32),
                       pltpu.VMEM((_SLOTS, nt, 8, 128), jnp.float32),
                       pltpu.VMEM((K, bpw // _PHASES), jnp.float32),
                       pltpu.SemaphoreType.DMA],
        compiler_params=pltpu.CompilerParams(needs_layout_passes=False),
    )
    def gather(rows, tab, out, idxv, stage, outv, sem):
        wid = lax.axis_index("s") * _NC + lax.axis_index("c")
        base = wid * bpw
        lane = jnp.arange(16, dtype=jnp.int32)
        pltpu.sync_copy(rows.at[pl.ds(base, bpw)], idxv)

        def scalar_idx(j):
            v16 = idxv[pl.ds((j >> 4) << 4, 16)]
            return lax.reduce_max(jnp.where(lane == (j & 15), v16, 0),
                                  axes=(0,))

        def issue(s, tc):
            c0 = pl.multiple_of(tc << 7, 128)
            for t in range(nt):
                pltpu.async_copy(tab.at[pl.ds(t * 8, 8), pl.ds(c0, 128)],
                                 stage.at[s, t], sem)

        def drain():
            for t in range(nt):
                pltpu.make_async_copy(
                    tab.at[pl.ds(0, 8), pl.ds(0, 128)],
                    stage.at[0, 0], sem).wait()

        half = bpw // _PHASES

        def extract(j, s, r):
            rcol = jnp.full((16,), r & 127, jnp.int32)
            col = jnp.full((16,), j & (half - 1), jnp.int32)
            sv = jnp.full((16,), s, jnp.int32)
            for cc in range(K // 16):
                kv = lane + 16 * cc
                va = plsc.load_gather(stage, [sv, kv >> 3, kv & 7, rcol])
                plsc.store_scatter(outv, [kv, col], va)

        def mod_slots(s):
            return s - (s // _SLOTS) * _SLOTS

        def writeback(phase):
            def wb(k, c):
                pltpu.sync_copy(outv.at[k],
                                out.at[pl.ds(k * B + base + phase * half, half)])
                return c
            lax.fori_loop(0, K, wb, 0)

        chain = []
        r_p, s_p, f_p = None, jnp.int32(0), None
        for j in range(_DEPTH):
            r_j = scalar_idx(j)
            if j == 0:
                f_j, s_j = jnp.bool_(True), jnp.int32(0)
            else:
                f_j = (r_j >> 7) != (r_p >> 7)
                s_j = jnp.where(f_j, mod_slots(s_p + 1), s_p)
            lax.cond(f_j, lambda r=r_j, s=s_j: issue(s, r >> 7), lambda: None)
            chain.insert(0, (r_j, s_j, f_j))
            r_p, s_p, f_p = r_j, s_j, f_j

        def body(j, carry):
            rh, sh, fh = carry[0:3]
            r0, s0, f0 = carry[-3:]
            rn = scalar_idx(j + _DEPTH)
            fn = (rn >> 7) != (rh >> 7)
            sn = jnp.where(fn, mod_slots(sh + 1), sh)
            lax.cond(fn, lambda: issue(sn, rn >> 7), lambda: None)
            lax.cond(f0, drain, lambda: None)
            extract(j, s0, r0)
            for p in range(_PHASES - 1):
                lax.cond(j == (p + 1) * half - 1,
                         lambda p=p: writeback(p), lambda: None)
            return (rn, sn, fn) + carry[:-3]

        init = tuple(x for t in chain for x in t)
        carry = lax.fori_loop(0, bpw - _DEPTH, body, init)
        tail = [carry[3 * i:3 * i + 3] for i in reversed(range(_DEPTH))]
        for i, (r_j, s_j, f_j) in enumerate(tail):
            lax.cond(f_j, drain, lambda: None)
            extract(bpw - _DEPTH + i, s_j, r_j)
        writeback(_PHASES - 1)

    return gather


def _dense_body(gut_ref, git_ref, epst_ref, gr_ref, gr3_ref, lvt_ref, xui_ref):
    gut = gut_ref[...]
    git = git_ref[...]
    gr = gr_ref[...]
    w = jnp.exp(0.5 * gr)
    scores = lax.dot_general(w, gut, (((1,), (0,)), ((), ())),
                             preferred_element_type=jnp.float32)
    m = jnp.max(scores, axis=0, keepdims=True)
    ex = jnp.exp(scores - m)
    s = ex / jnp.sum(ex, axis=0, keepdims=True)
    lvt = s[:, None, :] * gr3_ref[...]
    lvt_ref[...] = lvt
    noise = epst_ref[...] * jnp.exp(0.5 * lvt)
    gudot = jnp.sum(gut * git, axis=0)
    nd = jnp.sum(noise * git[None, :, :], axis=1)
    ands = nd + gudot[None, :]
    p = 1.0 - jax.nn.sigmoid(ands) + _EPS_C
    xui_ref[...] = 1.0 - jnp.exp(jnp.sum(jnp.log(p), axis=0))


@functools.lru_cache(maxsize=2)
def _dense_fn(B: int, K: int, blk: int, interpret: bool = False):
    grid = (B // blk,)
    return pl.pallas_call(
        _dense_body,
        grid=grid,
        in_specs=[
            pl.BlockSpec((K, blk), lambda i: (0, i)),
            pl.BlockSpec((K, blk), lambda i: (0, i)),
            pl.BlockSpec((_N_RULES, K, blk), lambda i: (0, 0, i)),
            pl.BlockSpec((_N_RULES, K), lambda i: (0, 0)),
            pl.BlockSpec((_N_RULES, K, 1), lambda i: (0, 0, 0)),
        ],
        out_specs=[
            pl.BlockSpec((_N_RULES, K, blk), lambda i: (0, 0, i)),
            pl.BlockSpec((blk,), lambda i: (i,)),
        ],
        out_shape=[
            jax.ShapeDtypeStruct((_N_RULES, K, B), jnp.float32),
            jax.ShapeDtypeStruct((B,), jnp.float32),
        ],
        interpret=interpret,
    )


def kernel(users, items, Gu_mean, Gr, Gi):
    B = users.shape[0]
    N, K = Gu_mean.shape
    users = users.astype(jnp.int32)
    items = items.astype(jnp.int32)
    iota = jnp.arange(B, dtype=jnp.int32)
    g = _gather_fn(B, K, N)
    outs = []
    for idx, tab in ((users, Gu_mean), (items, Gi)):
        s_idx, order = lax.sort_key_val(idx, iota)
        got = g(s_idx, tab.T).reshape(K, B)
        _, inv = lax.sort_key_val(order, iota)
        outs.append(jnp.take(got, inv, axis=1))
    gu_t, gi_t = outs
    eps_t = _eps_const(B, K)
    lvt, xui = _dense_fn(B, K, 512)(gu_t, gi_t, eps_t, Gr, Gr[:, :, None])
    gu = gu_t.T
    lv = jnp.transpose(lvt, (2, 0, 1))
    return xui, gu, lv
